# baseline (speedup 1.0000x reference)
import functools

import jax
import jax.numpy as jnp
from jax.experimental import pallas as pl
from jax.experimental.pallas import tpu as pltpu

HIDDEN = 2048
EXPERTS = 16
BT = 512


def _router_block(x_ref, wt_ref, logits_ref, probs_ref, ent_ref, tw_ref, ti_ref):
    xb = x_ref[...]
    wt = wt_ref[...]
    logits = jnp.dot(xb, wt, preferred_element_type=jnp.float32)
    logits_ref[...] = logits

    m = jnp.max(logits, axis=1, keepdims=True)
    e = jnp.exp(logits - m)
    s = jnp.sum(e, axis=1, keepdims=True)
    probs = e / s
    probs_ref[...] = probs

    ent_ref[...] = -jnp.sum(probs * jnp.log(probs + 1e-9), axis=1, keepdims=True)

    cols = jax.lax.broadcasted_iota(jnp.int32, (BT, EXPERTS), 1)
    w1 = jnp.max(probs, axis=1, keepdims=True)
    i1 = jnp.min(jnp.where(probs == w1, cols, EXPERTS), axis=1, keepdims=True)
    masked = jnp.where(cols == i1, -jnp.inf, probs)
    w2 = jnp.max(masked, axis=1, keepdims=True)
    i2 = jnp.min(jnp.where(masked == w2, cols, EXPERTS), axis=1, keepdims=True)

    tot = w1 + w2 + 1e-9
    tw_ref[...] = jnp.concatenate([w1 / tot, w2 / tot], axis=1)
    ti_ref[...] = jnp.concatenate([i1, i2], axis=1)


@functools.partial(jax.jit, static_argnames=())
def kernel(x, W):
    b, s, h = x.shape
    T = b * s
    x_flat = x.reshape(T, h)
    wt = W.T

    grid = (T // BT,)
    out_shapes = (
        jax.ShapeDtypeStruct((T, EXPERTS), jnp.float32),
        jax.ShapeDtypeStruct((T, EXPERTS), jnp.float32),
        jax.ShapeDtypeStruct((T, 1), jnp.float32),
        jax.ShapeDtypeStruct((T, 2), jnp.float32),
        jax.ShapeDtypeStruct((T, 2), jnp.int32),
    )
    tok_spec = lambda w: pl.BlockSpec((BT, w), lambda i: (i, 0))
    logits, probs, ent, tw, ti = pl.pallas_call(
        _router_block,
        grid=grid,
        in_specs=[
            pl.BlockSpec((BT, HIDDEN), lambda i: (i, 0)),
            pl.BlockSpec((HIDDEN, EXPERTS), lambda i: (0, 0)),
        ],
        out_specs=(
            tok_spec(EXPERTS),
            tok_spec(EXPERTS),
            tok_spec(1),
            tok_spec(2),
            tok_spec(2),
        ),
        out_shape=out_shapes,
        compiler_params=pltpu.CompilerParams(
            dimension_semantics=("arbitrary",),
        ),
    )(x_flat, wt)

    entropy = ent.reshape(T)
    return (tw, ti, probs, probs, logits, entropy)
